# baseline (speedup 1.0000x reference)
import functools

import jax
import jax.numpy as jnp
import numpy as np
from jax import lax
from jax.experimental import pallas as pl
from jax.experimental.pallas import tpu as pltpu
from jax.experimental.pallas import tpu_sc as plsc

N_NODES = 10000
N_EDGES = 320000
D = 128
H = D // 2

NC = 2
NS = 16

CHUNK = 80
CHUNKS_PER_TILE = N_EDGES // (NS * CHUNK)

NP = 10240
ROWS_PER_TILE = NP // NS

BM = 400

_PERM = np.empty(D, dtype=np.int32)
for _h in range(4):
    _base = 32 * _h
    for _k in range(16):
        _PERM[_base + 2 * _k] = _base + _k
        _PERM[_base + 2 * _k + 1] = _base + 16 + _k


def _mm_body(x_ref, w1_ref, w2_ref, b_ref, t_ref):
    xb = x_ref[...]
    m1 = jnp.dot(xb, w1_ref[...], preferred_element_type=jnp.float32) + b_ref[...]
    m2 = jnp.dot(xb, w2_ref[...], preferred_element_type=jnp.float32)
    t_ref[0] = m1[:, :H].astype(jnp.bfloat16)
    t_ref[1] = m1[:, H:].astype(jnp.bfloat16)
    t_ref[2] = m2[:, :H].astype(jnp.bfloat16)
    t_ref[3] = m2[:, H:].astype(jnp.bfloat16)


def _precompute_table(x, w1, w2, b2d):
    return pl.pallas_call(
        _mm_body,
        grid=(N_NODES // BM,),
        in_specs=[
            pl.BlockSpec((BM, D), lambda i: (i, 0)),
            pl.BlockSpec((D, D), lambda i: (0, 0)),
            pl.BlockSpec((D, D), lambda i: (0, 0)),
            pl.BlockSpec((1, D), lambda i: (0, 0)),
        ],
        out_specs=pl.BlockSpec((4, BM, H), lambda i: (0, i, 0)),
        out_shape=jax.ShapeDtypeStruct((4, N_NODES, H), jnp.bfloat16),
    )(x, w1, w2, b2d)


def _combine_body(p_ref, o_ref):
    o_ref[:, :H] = p_ref[0]
    o_ref[:, H:] = p_ref[1]


def _combine(partials):
    return pl.pallas_call(
        _combine_body,
        grid=(N_NODES // BM,),
        in_specs=[pl.BlockSpec((NC, BM, H), lambda i: (0, i, 0))],
        out_specs=pl.BlockSpec((BM, D), lambda i: (i, 0)),
        out_shape=jax.ShapeDtypeStruct((N_NODES, D), jnp.float32),
    )(partials)


@functools.partial(
    pl.kernel,
    out_type=jax.ShapeDtypeStruct((NC, NP, H), jnp.float32),
    mesh=plsc.VectorSubcoreMesh(core_axis_name="c", subcore_axis_name="s"),
    scratch_types=[
        pltpu.VMEM((CHUNKS_PER_TILE, CHUNK), jnp.int32),
        pltpu.VMEM((CHUNKS_PER_TILE, CHUNK), jnp.int32),
        pltpu.VMEM((CHUNK,), jnp.int32),
        pltpu.VMEM((CHUNK, H), jnp.bfloat16),
        pltpu.VMEM((CHUNK, H), jnp.bfloat16),
        pltpu.VMEM((CHUNK, H), jnp.float32),
        pltpu.VMEM_SHARED((NP, H), jnp.float32),
        pltpu.SemaphoreType.DMA,
        pltpu.SemaphoreType.DMA,
    ],
    compiler_params=pltpu.CompilerParams(
        use_tc_tiling_on_sc=False, needs_layout_passes=False
    ),
)
def _sc_edges(t_hbm, srcg_hbm, dstg_hbm, out_hbm,
              sidx, didx, kidx, ra, rb, msg, accum, sem_a, sem_b):
    c = lax.axis_index("c")
    s = lax.axis_index("s")
    boff = (2 + c) * N_NODES

    zero = jnp.zeros((16,), jnp.float32)

    @pl.loop(0, CHUNK)
    def _zero_rows(e):
        for j in range(H // 16):
            msg[e, pl.ds(j * 16, 16)] = zero

    row0 = s * ROWS_PER_TILE

    @pl.loop(0, ROWS_PER_TILE // CHUNK)
    def _zero_accum(k):
        pltpu.sync_copy(msg, accum.at[pl.ds(row0 + k * CHUNK, CHUNK)])

    pltpu.sync_copy(srcg_hbm.at[c, s], sidx)
    pltpu.sync_copy(dstg_hbm.at[c, s], didx)

    plsc.subcore_barrier()

    himask = jnp.full((16,), -65536, jnp.int32)

    @pl.loop(0, CHUNKS_PER_TILE)
    def _chunk(g):
        cp_a = pltpu.async_copy(t_hbm.at[sidx.at[g]], ra, sem_a)
        cp_b = pltpu.async_copy(t_hbm.at[didx.at[g]], rb, sem_b)

        for j in range(CHUNK // 16):
            sl = pl.ds(j * 16, 16)
            kidx[sl] = didx[g, sl] - boff

        cp_a.wait()
        cp_b.wait()

        @pl.loop(0, CHUNK, unroll=2)
        def _row(e):
            for j in range(H // 32):
                a32 = plsc.bitcast(ra[e, pl.ds(j * 32, 32)], jnp.int32)
                b32 = plsc.bitcast(rb[e, pl.ds(j * 32, 32)], jnp.int32)
                ae = plsc.bitcast(a32 << 16, jnp.float32)
                be = plsc.bitcast(b32 << 16, jnp.float32)
                ao = plsc.bitcast(a32 & himask, jnp.float32)
                bo = plsc.bitcast(b32 & himask, jnp.float32)
                msg[e, pl.ds(j * 32, 16)] = jnp.maximum(ae + be, 0.0)
                msg[e, pl.ds(j * 32 + 16, 16)] = jnp.maximum(ao + bo, 0.0)

        pltpu.sync_copy(msg, accum.at[kidx], add=True)

    plsc.subcore_barrier()
    pltpu.sync_copy(
        accum.at[pl.ds(row0, ROWS_PER_TILE)],
        out_hbm.at[c, pl.ds(row0, ROWS_PER_TILE)],
    )


def kernel(x, edge_index, W, b):
    perm = jnp.asarray(_PERM)
    wp = W[:, perm]
    w1 = wp[:D]
    w2 = wp[D:]
    b2d = b[perm].reshape(1, D)
    table = _precompute_table(x, w1, w2, b2d).reshape(4 * N_NODES, H)
    src = edge_index[0].reshape(NS, CHUNKS_PER_TILE, CHUNK)
    dst = edge_index[1].reshape(NS, CHUNKS_PER_TILE, CHUNK)
    srcg = jnp.stack([src, src + N_NODES])
    dstg = jnp.stack([dst + 2 * N_NODES, dst + 3 * N_NODES])
    partials = _sc_edges(table, srcg, dstg)
    return _combine(partials)
